# baseline (speedup 1.0000x reference)
import functools

import numpy as np
import jax
import jax.numpy as jnp
from jax import lax
from jax.experimental import pallas as pl
from jax.experimental.pallas import tpu as pltpu
from jax.experimental.pallas import tpu_sc as plsc

TOPK = 8
TEMP = 3.0
R = 128
V = 100000
WB = 8960
NB = -(-V // WB)
LN = 128

_NC = 2
_NS = 16
_NW = _NC * _NS
_EPW = (R * TOPK) // _NW


_NCOL = WB // LN


def _cx(av, ai, bv, bi):
    take = av >= bv
    return (jnp.where(take, av, bv), jnp.where(take, ai, bi),
            jnp.where(take, bv, av), jnp.where(take, bi, ai))


def _bitonic_merge(vs, ii):
    n = len(vs)
    d = n // 2
    while d >= 1:
        for k in range(n):
            if (k % (2 * d)) < d:
                hv, hi_, lv, li = _cx(vs[k], ii[k], vs[k + d], ii[k + d])
                vs[k], ii[k], vs[k + d], ii[k + d] = hv, hi_, lv, li
        d //= 2
    return vs, ii


def _merge_lists(A, B):
    av, ai = A
    bv, bi = B
    while len(bv) < len(av):
        bv = bv + [jnp.full_like(bv[0], -jnp.inf)]
        bi = bi + [jnp.zeros_like(bi[0])]
    while len(av) < len(bv):
        av = av + [jnp.full_like(av[0], -jnp.inf)]
        ai = ai + [jnp.zeros_like(ai[0])]
    L = len(av)
    if L < TOPK:
        vs, ii = _bitonic_merge(av + bv[::-1], ai + bi[::-1])
        return vs, ii
    vs, ii = [], []
    for k in range(TOPK):
        take = av[k] >= bv[TOPK - 1 - k]
        vs.append(jnp.where(take, av[k], bv[TOPK - 1 - k]))
        ii.append(jnp.where(take, ai[k], bi[TOPK - 1 - k]))
    return _bitonic_merge(vs, ii)


def _topk_body(t_ref, vals_out, idx_out, vals_s, idx_s):
    j = pl.program_id(0)

    @pl.when(j == 0)
    def _init():
        vals_s[...] = jnp.full((R, TOPK), -jnp.inf, jnp.float32)
        idx_s[...] = jnp.zeros((R, TOPK), jnp.int32)

    base = j * WB
    lane = lax.broadcasted_iota(jnp.int32, (R, LN), 1)

    def tree(c0, c1):
        if c1 - c0 == 1:
            gi = lane + (base + c0 * LN)
            xc = t_ref[:, c0 * LN:(c0 + 1) * LN]
            xc = jnp.where(gi < V, xc, -jnp.inf)
            return ([xc], [gi])
        mid = (c0 + c1) // 2
        return _merge_lists(tree(c0, mid), tree(mid, c1))

    tv, ti = tree(0, _NCOL)

    mv = jnp.concatenate(tv + [vals_s[...]], axis=1)
    mi = jnp.concatenate(ti + [idx_s[...]], axis=1)
    nv = []
    ni = []
    for _ in range(TOPK):
        m = jnp.max(mv, axis=1, keepdims=True)
        hit = mv == m
        sel = jnp.min(jnp.where(hit, mi, V), axis=1, keepdims=True)
        nv.append(m)
        ni.append(sel)
        mv = jnp.where(hit & (mi == sel), -jnp.inf, mv)
    vals_s[...] = jnp.concatenate(nv, axis=1)
    idx_s[...] = jnp.concatenate(ni, axis=1)

    @pl.when(j == NB - 1)
    def _fin():
        vals_out[...] = vals_s[...]
        idx_out[...] = idx_s[...]


_topk_call = pl.pallas_call(
    _topk_body,
    grid=(NB,),
    in_specs=[pl.BlockSpec((R, WB), lambda j: (0, j))],
    out_specs=[pl.BlockSpec((R, TOPK), lambda j: (0, 0)),
               pl.BlockSpec((R, TOPK), lambda j: (0, 0))],
    out_shape=[jax.ShapeDtypeStruct((R, TOPK), jnp.float32),
               jax.ShapeDtypeStruct((R, TOPK), jnp.int32)],
    scratch_shapes=[pltpu.VMEM((R, TOPK), jnp.float32),
                    pltpu.VMEM((R, TOPK), jnp.int32)],
    compiler_params=pltpu.CompilerParams(
        dimension_semantics=("arbitrary",)),
)


@functools.partial(
    pl.kernel,
    out_type=jax.ShapeDtypeStruct((R * TOPK,), jnp.float32),
    mesh=plsc.VectorSubcoreMesh(core_axis_name="c", subcore_axis_name="s"),
    scratch_types=[
        pltpu.VMEM((_EPW,), jnp.int32),
        pltpu.VMEM((_EPW, 16), jnp.float32),
        pltpu.VMEM((_EPW,), jnp.float32),
        pltpu.SemaphoreType.DMA,
    ],
)
def _gather_call(s_hbm, idx_hbm, out_hbm, idxv, rows_v, outv, sem):
    c = lax.axis_index("c")
    s = lax.axis_index("s")
    w = s * _NC + c
    ebase = w * _EPW
    rbase = w * (_EPW // TOPK)
    pltpu.sync_copy(idx_hbm.at[pl.ds(ebase, _EPW)], idxv)

    vecs = [idxv[pl.ds(v * 16, 16)] for v in range(_EPW // 16)]
    copies = []
    for e in range(_EPW):
        pos_e = vecs[e // 16][e % 16]
        off_e = pl.multiple_of((pos_e // 16) * 16, 16)
        row_e = rbase + e // TOPK
        copies.append(pltpu.async_copy(
            s_hbm.at[row_e, pl.ds(off_e, 16)], rows_v.at[e], sem))
    for cp in copies:
        cp.wait()

    iota = lax.iota(jnp.int32, 16)
    accs = []
    for v in range(_EPW // 16):
        acc = jnp.zeros((16,), jnp.float32)
        for i in range(16):
            e = v * 16 + i
            lane_e = vecs[e // 16][e % 16] % 16
            chunk = rows_v[e]
            val = chunk[0]
            for l in range(1, 16):
                val = jnp.where(lane_e == l, chunk[l], val)
            acc = jnp.where(iota == i, val, acc)
        accs.append(acc)
    for v, acc in enumerate(accs):
        outv[pl.ds(v * 16, 16)] = acc
    pltpu.sync_copy(outv, out_hbm.at[pl.ds(ebase, _EPW)])


_PI, _PJ = np.triu_indices(TOPK, k=1)
NPAIR = len(_PI)
_M = np.zeros((TOPK, LN), np.float32)
for _p, (_a, _b) in enumerate(zip(_PI, _PJ)):
    _M[_a, _p] += 1.0 / TEMP
    _M[_b, _p] -= 1.0 / TEMP
_PMASK = np.zeros((1, LN), np.float32)
_PMASK[0, :NPAIR] = 1.0


def _kl_body(t_ref, s_ref, m_ref, mask_ref, out_ref):
    t = t_ref[...]
    sv = s_ref[...]
    mm = m_ref[...]
    mask = mask_ref[...] > 0
    d_t = jnp.dot(t, mm, preferred_element_type=jnp.float32)
    d_s = jnp.dot(sv, mm, preferred_element_type=jnp.float32)

    neg = jnp.float32(-jnp.inf)
    mt = jnp.max(jnp.where(mask, d_t, neg), axis=1, keepdims=True)
    et = jnp.where(mask, jnp.exp(d_t - mt), 0.0)
    st = jnp.sum(et, axis=1, keepdims=True)
    ms = jnp.max(jnp.where(mask, d_s, neg), axis=1, keepdims=True)
    es = jnp.where(mask, jnp.exp(d_s - ms), 0.0)
    ss = jnp.sum(es, axis=1, keepdims=True)

    log_pt = d_t - mt - jnp.log(st)
    log_ps = d_s - ms - jnp.log(ss)
    kl = jnp.where(mask, (et / st) * (log_pt - log_ps), 0.0)
    out_ref[...] = jnp.broadcast_to(jnp.sum(kl) / R, (1, 1))


_kl_call = pl.pallas_call(
    _kl_body,
    out_shape=jax.ShapeDtypeStruct((1, 1), jnp.float32),
)


@jax.jit
def kernel(logits_s, logits_t):
    t_vals, t_idx = _topk_call(logits_t)
    s_vals = _gather_call(logits_s, t_idx.reshape(-1)).reshape(R, TOPK)
    loss = _kl_call(t_vals, s_vals, jnp.asarray(_M), jnp.asarray(_PMASK))
    return loss.reshape(())


# speedup vs baseline: 1.0413x; 1.0413x over previous
import functools

import numpy as np
import jax
import jax.numpy as jnp
from jax import lax
from jax.experimental import pallas as pl
from jax.experimental.pallas import tpu as pltpu
from jax.experimental.pallas import tpu_sc as plsc

TOPK = 8
TEMP = 3.0
R = 128
V = 100000
WB = 8192
NB = -(-V // WB)
LN = 128

_NC = 2
_NS = 16
_NW = _NC * _NS
_EPW = (R * TOPK) // _NW


_NCOL = WB // LN


def _cx(av, ai, bv, bi):
    take = av >= bv
    return (jnp.where(take, av, bv), jnp.where(take, ai, bi),
            jnp.where(take, bv, av), jnp.where(take, bi, ai))


def _bitonic_merge(vs, ii):
    n = len(vs)
    d = n // 2
    while d >= 1:
        for k in range(n):
            if (k % (2 * d)) < d:
                hv, hi_, lv, li = _cx(vs[k], ii[k], vs[k + d], ii[k + d])
                vs[k], ii[k], vs[k + d], ii[k + d] = hv, hi_, lv, li
        d //= 2
    return vs, ii


def _merge_lists(A, B):
    av, ai = A
    bv, bi = B
    while len(bv) < len(av):
        bv = bv + [jnp.full_like(bv[0], -jnp.inf)]
        bi = bi + [jnp.zeros_like(bi[0])]
    while len(av) < len(bv):
        av = av + [jnp.full_like(av[0], -jnp.inf)]
        ai = ai + [jnp.zeros_like(ai[0])]
    L = len(av)
    if L < TOPK:
        vs, ii = _bitonic_merge(av + bv[::-1], ai + bi[::-1])
        return vs, ii
    vs, ii = [], []
    for k in range(TOPK):
        take = av[k] >= bv[TOPK - 1 - k]
        vs.append(jnp.where(take, av[k], bv[TOPK - 1 - k]))
        ii.append(jnp.where(take, ai[k], bi[TOPK - 1 - k]))
    return _bitonic_merge(vs, ii)


def _topk_body(t_ref, vals_out, idx_out, vals_s, idx_s):
    j = pl.program_id(0)

    @pl.when(j == 0)
    def _init():
        vals_s[...] = jnp.full((R, TOPK), -jnp.inf, jnp.float32)
        idx_s[...] = jnp.zeros((R, TOPK), jnp.int32)

    base = j * WB
    lane = lax.broadcasted_iota(jnp.int32, (R, LN), 1)

    def tree(c0, c1):
        if c1 - c0 == 1:
            gi = lane + (base + c0 * LN)
            xc = t_ref[:, c0 * LN:(c0 + 1) * LN]
            xc = jnp.where(gi < V, xc, -jnp.inf)
            return ([xc], [gi])
        mid = (c0 + c1) // 2
        return _merge_lists(tree(c0, mid), tree(mid, c1))

    tv, ti = tree(0, _NCOL)

    mv = jnp.concatenate(tv + [vals_s[...]], axis=1)
    mi = jnp.concatenate(ti + [idx_s[...]], axis=1)
    nv = []
    ni = []
    for _ in range(TOPK):
        m = jnp.max(mv, axis=1, keepdims=True)
        hit = mv == m
        sel = jnp.min(jnp.where(hit, mi, V), axis=1, keepdims=True)
        nv.append(m)
        ni.append(sel)
        mv = jnp.where(hit & (mi == sel), -jnp.inf, mv)
    vals_s[...] = jnp.concatenate(nv, axis=1)
    idx_s[...] = jnp.concatenate(ni, axis=1)

    @pl.when(j == NB - 1)
    def _fin():
        vals_out[...] = vals_s[...]
        idx_out[...] = idx_s[...]


_topk_call = pl.pallas_call(
    _topk_body,
    grid=(NB,),
    in_specs=[pl.BlockSpec((R, WB), lambda j: (0, j))],
    out_specs=[pl.BlockSpec((R, TOPK), lambda j: (0, 0)),
               pl.BlockSpec((R, TOPK), lambda j: (0, 0))],
    out_shape=[jax.ShapeDtypeStruct((R, TOPK), jnp.float32),
               jax.ShapeDtypeStruct((R, TOPK), jnp.int32)],
    scratch_shapes=[pltpu.VMEM((R, TOPK), jnp.float32),
                    pltpu.VMEM((R, TOPK), jnp.int32)],
    compiler_params=pltpu.CompilerParams(
        dimension_semantics=("arbitrary",)),
)


@functools.partial(
    pl.kernel,
    out_type=jax.ShapeDtypeStruct((R * TOPK,), jnp.float32),
    mesh=plsc.VectorSubcoreMesh(core_axis_name="c", subcore_axis_name="s"),
    scratch_types=[
        pltpu.VMEM((_EPW,), jnp.int32),
        pltpu.VMEM((_EPW, 16), jnp.float32),
        pltpu.VMEM((_EPW,), jnp.float32),
        pltpu.SemaphoreType.DMA,
    ],
)
def _gather_call(s_hbm, idx_hbm, out_hbm, idxv, rows_v, outv, sem):
    c = lax.axis_index("c")
    s = lax.axis_index("s")
    w = s * _NC + c
    ebase = w * _EPW
    rbase = w * (_EPW // TOPK)
    pltpu.sync_copy(idx_hbm.at[pl.ds(ebase, _EPW)], idxv)

    vecs = [idxv[pl.ds(v * 16, 16)] for v in range(_EPW // 16)]
    copies = []
    for e in range(_EPW):
        pos_e = vecs[e // 16][e % 16]
        off_e = pl.multiple_of((pos_e // 16) * 16, 16)
        row_e = rbase + e // TOPK
        copies.append(pltpu.async_copy(
            s_hbm.at[row_e, pl.ds(off_e, 16)], rows_v.at[e], sem))
    for cp in copies:
        cp.wait()

    iota = lax.iota(jnp.int32, 16)
    accs = []
    for v in range(_EPW // 16):
        acc = jnp.zeros((16,), jnp.float32)
        for i in range(16):
            e = v * 16 + i
            lane_e = vecs[e // 16][e % 16] % 16
            chunk = rows_v[e]
            val = chunk[0]
            for l in range(1, 16):
                val = jnp.where(lane_e == l, chunk[l], val)
            acc = jnp.where(iota == i, val, acc)
        accs.append(acc)
    for v, acc in enumerate(accs):
        outv[pl.ds(v * 16, 16)] = acc
    pltpu.sync_copy(outv, out_hbm.at[pl.ds(ebase, _EPW)])


_PI, _PJ = np.triu_indices(TOPK, k=1)
NPAIR = len(_PI)
_M = np.zeros((TOPK, LN), np.float32)
for _p, (_a, _b) in enumerate(zip(_PI, _PJ)):
    _M[_a, _p] += 1.0 / TEMP
    _M[_b, _p] -= 1.0 / TEMP
_PMASK = np.zeros((1, LN), np.float32)
_PMASK[0, :NPAIR] = 1.0


def _kl_body(t_ref, s_ref, m_ref, mask_ref, out_ref):
    t = t_ref[...]
    sv = s_ref[...]
    mm = m_ref[...]
    mask = mask_ref[...] > 0
    d_t = jnp.dot(t, mm, preferred_element_type=jnp.float32)
    d_s = jnp.dot(sv, mm, preferred_element_type=jnp.float32)

    neg = jnp.float32(-jnp.inf)
    mt = jnp.max(jnp.where(mask, d_t, neg), axis=1, keepdims=True)
    et = jnp.where(mask, jnp.exp(d_t - mt), 0.0)
    st = jnp.sum(et, axis=1, keepdims=True)
    ms = jnp.max(jnp.where(mask, d_s, neg), axis=1, keepdims=True)
    es = jnp.where(mask, jnp.exp(d_s - ms), 0.0)
    ss = jnp.sum(es, axis=1, keepdims=True)

    log_pt = d_t - mt - jnp.log(st)
    log_ps = d_s - ms - jnp.log(ss)
    kl = jnp.where(mask, (et / st) * (log_pt - log_ps), 0.0)
    out_ref[...] = jnp.broadcast_to(jnp.sum(kl) / R, (1, 1))


_kl_call = pl.pallas_call(
    _kl_body,
    out_shape=jax.ShapeDtypeStruct((1, 1), jnp.float32),
)


@jax.jit
def kernel(logits_s, logits_t):
    t_vals, t_idx = _topk_call(logits_t)
    s_vals = _gather_call(logits_s, t_idx.reshape(-1)).reshape(R, TOPK)
    loss = _kl_call(t_vals, s_vals, jnp.asarray(_M), jnp.asarray(_PMASK))
    return loss.reshape(())
